# Initial kernel scaffold; baseline (speedup 1.0000x reference)
import jax
import jax.numpy as jnp
from jax.experimental import pallas as pl


def kernel(batch, weight):
    raise NotImplementedError("write your pallas kernel here")


# speedup vs baseline: 2.7596x; 2.7596x over previous
import functools

import jax
import jax.numpy as jnp
from jax import lax
from jax.experimental import pallas as pl
from jax.experimental.pallas import tpu as pltpu
from jax.experimental.pallas import tpu_sc as plsc

VOCAB = 1000
DIM = 32
ROWS = 16384 * 50
IDXW = 128
NC, NS = 2, 16
NW = NC * NS
ROWS_PER_W = ROWS // NW
IDX_ROWS_PER_W = ROWS_PER_W // IDXW
K = 20
MACROS = IDX_ROWS_PER_W // K


def _make_sc_gather():
    mesh = plsc.VectorSubcoreMesh(core_axis_name="c", subcore_axis_name="s")

    @functools.partial(
        pl.kernel,
        mesh=mesh,
        compiler_params=pltpu.CompilerParams(use_tc_tiling_on_sc=False),
        out_type=jax.ShapeDtypeStruct((ROWS, DIM), jnp.float32),
        scratch_types=[
            pltpu.VMEM((IDX_ROWS_PER_W, IDXW), jnp.int32),
            pltpu.VMEM((K * IDXW, DIM), jnp.float32),
            pltpu.SemaphoreType.DMA,
        ],
    )
    def k(table_hbm, idx_hbm, out_hbm, idx_v, rows_v, sem):
        wid = lax.axis_index("s") * NC + lax.axis_index("c")
        idx_row0 = wid * IDX_ROWS_PER_W
        pltpu.sync_copy(idx_hbm.at[pl.ds(idx_row0, IDX_ROWS_PER_W)], idx_v)

        def macro(m, carry):
            r0 = m * K
            cps = [
                pltpu.async_copy(
                    table_hbm.at[idx_v.at[r0 + j]],
                    rows_v.at[pl.ds(j * IDXW, IDXW)],
                    sem,
                )
                for j in range(K)
            ]
            for cp in cps:
                cp.wait()
            pltpu.sync_copy(
                rows_v,
                out_hbm.at[pl.ds((idx_row0 + r0) * IDXW, K * IDXW)],
            )
            return carry

        lax.fori_loop(0, MACROS, macro, 0)

    return k


_sc_gather = _make_sc_gather()


def kernel(batch, weight):
    b, h = batch.shape
    idx2d = batch.reshape(ROWS // IDXW, IDXW).astype(jnp.int32)
    flat = _sc_gather(weight, idx2d)
    return flat.reshape(b, h, DIM)


# speedup vs baseline: 2.7705x; 1.0039x over previous
import functools

import jax
import jax.numpy as jnp
from jax import lax
from jax.experimental import pallas as pl
from jax.experimental.pallas import tpu as pltpu
from jax.experimental.pallas import tpu_sc as plsc

VOCAB = 1000
DIM = 32
ROWS = 16384 * 50
IDXW = 128
NC, NS = 2, 16
NW = NC * NS
ROWS_PER_W = ROWS // NW
IDX_ROWS_PER_W = ROWS_PER_W // IDXW
K = 10
MACROS = IDX_ROWS_PER_W // K
CHUNK = K * IDXW


def _make_sc_gather():
    mesh = plsc.VectorSubcoreMesh(core_axis_name="c", subcore_axis_name="s")

    @functools.partial(
        pl.kernel,
        mesh=mesh,
        compiler_params=pltpu.CompilerParams(use_tc_tiling_on_sc=False),
        out_type=jax.ShapeDtypeStruct((ROWS, DIM), jnp.float32),
        scratch_types=[
            pltpu.VMEM((IDX_ROWS_PER_W, IDXW), jnp.int32),
            pltpu.VMEM((CHUNK, DIM), jnp.float32),
            pltpu.VMEM((CHUNK, DIM), jnp.float32),
            pltpu.SemaphoreType.DMA,
            pltpu.SemaphoreType.DMA,
        ],
    )
    def k(table_hbm, idx_hbm, out_hbm, idx_v, rows0_v, rows1_v, sem_g, sem_o):
        wid = lax.axis_index("s") * NC + lax.axis_index("c")
        idx_row0 = wid * IDX_ROWS_PER_W
        bufs = (rows0_v, rows1_v)
        pltpu.sync_copy(idx_hbm.at[pl.ds(idx_row0, IDX_ROWS_PER_W)], idx_v)

        def gather_into(m, buf):
            r0 = m * K
            cps = [
                pltpu.async_copy(
                    table_hbm.at[idx_v.at[r0 + j]],
                    buf.at[pl.ds(j * IDXW, IDXW)],
                    sem_g,
                )
                for j in range(K)
            ]
            for cp in cps:
                cp.wait()

        def start_write(m, buf):
            pltpu.async_copy(
                buf, out_hbm.at[pl.ds((idx_row0 + m * K) * IDXW, CHUNK)], sem_o
            )

        def wait_write(buf):
            pltpu.make_async_copy(
                buf, out_hbm.at[pl.ds(idx_row0 * IDXW, CHUNK)], sem_o
            ).wait()

        gather_into(0, bufs[0])
        start_write(0, bufs[0])
        gather_into(1, bufs[1])
        start_write(1, bufs[1])

        def macro(mm, carry):
            m = 2 * mm
            wait_write(rows0_v)
            gather_into(m, rows0_v)
            start_write(m, rows0_v)
            wait_write(rows1_v)
            gather_into(m + 1, rows1_v)
            start_write(m + 1, rows1_v)
            return carry

        lax.fori_loop(1, MACROS // 2, macro, 0, unroll=False)

        wait_write(rows0_v)
        wait_write(rows1_v)

    return k


_sc_gather = _make_sc_gather()


def kernel(batch, weight):
    b, h = batch.shape
    idx2d = batch.reshape(ROWS // IDXW, IDXW).astype(jnp.int32)
    flat = _sc_gather(weight, idx2d)
    return flat.reshape(b, h, DIM)
